# baseline (speedup 1.0000x reference)
import functools

import jax
import jax.numpy as jnp
from jax import lax
from jax.experimental import pallas as pl
from jax.experimental.pallas import tpu as pltpu
from jax.experimental.pallas import tpu_sc as plsc

N = 50000
E = 800000
C = 64
HC = 32
H = 256
NC, NS = 2, 16
BLK = 128
STR = 3
CHUNK = BLK * STR
PAIR = 2 * CHUNK
PROWS = PAIR // BLK
NBODY = 66
EPT = NBODY * PAIR
EPAD = EPT * NS
IDXROWS = EPAD // BLK
IDXAL = IDXROWS + 8
ROWS_PT = IDXROWS // NS
ACC_ROWS = 50048
APT = ACC_ROWS // NS
NOUT = ACC_ROWS
WPT = NOUT // NS


def _sc_segsum(xflat, gidx4, sidx2):
    mesh = plsc.VectorSubcoreMesh(core_axis_name="c", subcore_axis_name="s")

    @functools.partial(
        pl.kernel,
        out_type=jax.ShapeDtypeStruct((2, 2 * NOUT, HC), jnp.float32),
        mesh=mesh,
        scratch_types=[
            pltpu.VMEM_SHARED((ACC_ROWS, HC), jnp.float32),
            pltpu.VMEM((CHUNK, HC), jnp.float32),
            pltpu.VMEM((CHUNK, HC), jnp.float32),
            pltpu.VMEM((PROWS, BLK), jnp.int32),
            pltpu.VMEM((PROWS, BLK), jnp.int32),
            pltpu.VMEM((PROWS, BLK), jnp.int32),
            pltpu.VMEM((PROWS, BLK), jnp.int32),
            pltpu.SemaphoreType.DMA,
            pltpu.SemaphoreType.DMA,
            pltpu.SemaphoreType.DMA,
            pltpu.SemaphoreType.DMA,
        ],
        compiler_params=pltpu.CompilerParams(use_tc_tiling_on_sc=False),
    )
    def seg_kernel(x_hbm, g_hbm4, s_hbm2, out_hbm,
                   acc, rowsA, rowsB, g0, s0, g1, s1,
                   gsem, ssemA, ssemB, isem):
        c = lax.axis_index("c")
        s = lax.axis_index("s")

        for d in range(2):
            g_hbm = g_hbm4.at[d].at[c]
            s_hbm = s_hbm2.at[d]

            def idx_row0(b):
                return s * ROWS_PT + b * PROWS

            def fire_idx(b, gbuf, sbuf):
                pltpu.async_copy(g_hbm.at[pl.ds(idx_row0(b), PROWS)],
                                 gbuf, isem)
                pltpu.async_copy(s_hbm.at[pl.ds(idx_row0(b), PROWS)],
                                 sbuf, isem)

            def wait_idx(b, gbuf, sbuf):
                pltpu.make_async_copy(
                    g_hbm.at[pl.ds(idx_row0(b), PROWS)], gbuf, isem).wait()
                pltpu.make_async_copy(
                    s_hbm.at[pl.ds(idx_row0(b), PROWS)], sbuf, isem).wait()

            def drain_sb(sbuf):
                for u in range(STR):
                    pltpu.make_async_copy(
                        rowsB.at[pl.ds(u * BLK, BLK)],
                        acc.at[sbuf.at[STR + u]], ssemB).wait()

            def run_body(gbuf, sbuf):
                ga = [
                    pltpu.async_copy(x_hbm.at[gbuf.at[u]],
                                     rowsA.at[pl.ds(u * BLK, BLK)], gsem)
                    for u in range(STR)
                ]
                gb = [
                    pltpu.async_copy(x_hbm.at[gbuf.at[STR + u]],
                                     rowsB.at[pl.ds(u * BLK, BLK)], gsem)
                    for u in range(STR)
                ]
                for dd in ga:
                    dd.wait()
                sa = [
                    pltpu.async_copy(rowsA.at[pl.ds(u * BLK, BLK)],
                                     acc.at[sbuf.at[u]], ssemA, add=True)
                    for u in range(STR)
                ]
                for dd in gb:
                    dd.wait()
                for dd in sa:
                    dd.wait()
                for u in range(STR):
                    pltpu.async_copy(rowsB.at[pl.ds(u * BLK, BLK)],
                                     acc.at[sbuf.at[STR + u]],
                                     ssemB, add=True)

            def zrow(i, z):
                rowsA[i, pl.ds(0, 16)] = jnp.zeros((16,), jnp.float32)
                rowsA[i, pl.ds(16, 16)] = jnp.zeros((16,), jnp.float32)
                return z
            lax.fori_loop(0, CHUNK, zrow, 0)
            zbase = s * APT
            zdescs = []
            zoff = 0
            while zoff < APT:
                zn = min(CHUNK, APT - zoff)
                zdescs.append(pltpu.async_copy(
                    rowsA.at[pl.ds(0, zn)],
                    acc.at[pl.ds(zbase + zoff, zn)], gsem))
                zoff += zn
            for dd in zdescs:
                dd.wait()
            plsc.subcore_barrier()

            fire_idx(0, g0, s0)

            def body(tt, carry):
                b0 = 2 * tt
                wait_idx(b0, g0, s0)

                @pl.when(tt > 0)
                def _():
                    drain_sb(s1)

                fire_idx(b0 + 1, g1, s1)
                run_body(g0, s0)
                wait_idx(b0 + 1, g1, s1)
                drain_sb(s0)
                fire_idx(b0 + 2, g0, s0)
                run_body(g1, s1)
                return carry
            lax.fori_loop(0, NBODY // 2, body, 0)
            wait_idx(NBODY, g0, s0)
            drain_sb(s1)
            plsc.subcore_barrier()

            pltpu.sync_copy(
                acc.at[pl.ds(s * WPT, WPT)],
                out_hbm.at[d].at[pl.ds(c * NOUT + s * WPT, WPT)],
            )
            plsc.subcore_barrier()

    return seg_kernel(xflat, gidx4, sidx2)


def _mlp(out4, W1r, b1, W2, b2):
    B = 2000

    def body(a_ref, w1_ref, b1_ref, w2_ref, b2_ref, o_ref):
        h1 = (
            jnp.dot(a_ref[0, 0], w1_ref[0, 0], preferred_element_type=jnp.float32)
            + jnp.dot(a_ref[0, 1], w1_ref[0, 1], preferred_element_type=jnp.float32)
            + jnp.dot(a_ref[1, 0], w1_ref[1, 0], preferred_element_type=jnp.float32)
            + jnp.dot(a_ref[1, 1], w1_ref[1, 1], preferred_element_type=jnp.float32)
            + b1_ref[...]
        )
        h1 = jnp.maximum(h1, 0.0)
        o_ref[...] = (
            jnp.dot(h1, w2_ref[...], preferred_element_type=jnp.float32)
            + b2_ref[...]
        )

    return pl.pallas_call(
        body,
        grid=(N // B,),
        in_specs=[
            pl.BlockSpec((2, 2, B, HC), lambda i: (0, 0, i, 0)),
            pl.BlockSpec((2, 2, HC, H), lambda i: (0, 0, 0, 0)),
            pl.BlockSpec((1, H), lambda i: (0, 0)),
            pl.BlockSpec((H, C), lambda i: (0, 0)),
            pl.BlockSpec((1, C), lambda i: (0, 0)),
        ],
        out_specs=pl.BlockSpec((B, C), lambda i: (i, 0)),
        out_shape=jax.ShapeDtypeStruct((N, C), jnp.float32),
    )(out4, W1r, b1.reshape(1, H), W2, b2.reshape(1, C))


def kernel(x, edge_index, W1, b1, W2, b2):
    src = edge_index[0].astype(jnp.int32)
    dst = edge_index[1].astype(jnp.int32)
    npad = IDXAL * BLK - E
    gpad = jnp.zeros((npad,), jnp.int32)
    spad = jnp.full((npad,), N, jnp.int32)
    gsd = jnp.stack([jnp.concatenate([src, gpad]),
                     jnp.concatenate([dst, gpad])])
    gidx4 = (2 * gsd[:, None, :]
             + jnp.arange(2, dtype=jnp.int32)[None, :, None]
             ).reshape(2, 2, IDXAL, BLK)
    sidx2 = jnp.stack([jnp.concatenate([dst, spad]),
                       jnp.concatenate([src, spad])]).reshape(2, IDXAL, BLK)
    xflat = x.reshape(2 * N, HC)
    out = _sc_segsum(xflat, gidx4, sidx2)
    out4 = out.reshape(2, 2, NOUT, HC)
    return _mlp(out4, W1.reshape(2, 2, HC, H), b1, W2, b2)


# speedup vs baseline: 1.1029x; 1.1029x over previous
import functools

import jax
import jax.numpy as jnp
from jax import lax
from jax.experimental import pallas as pl
from jax.experimental.pallas import tpu as pltpu
from jax.experimental.pallas import tpu_sc as plsc

N = 50000
E = 800000
C = 64
HC = 32
H = 256
NC, NS = 2, 16
BLK = 128
STR = 3
CHUNK = BLK * STR
PAIR = 2 * CHUNK
PROWS = PAIR // BLK
NBODY = 66
EPT = NBODY * PAIR
EPAD = EPT * NS
IDXROWS = EPAD // BLK
IDXAL = IDXROWS + 8
ROWS_PT = IDXROWS // NS
ACC_ROWS = 50048
APT = ACC_ROWS // NS
NOUT = ACC_ROWS
WPT = NOUT // NS


def _sc_segsum(xflat, gidx4, sidx2):
    mesh = plsc.VectorSubcoreMesh(core_axis_name="c", subcore_axis_name="s")

    @functools.partial(
        pl.kernel,
        out_type=jax.ShapeDtypeStruct((2, 2 * NOUT, HC), jnp.float32),
        mesh=mesh,
        scratch_types=[
            pltpu.VMEM_SHARED((ACC_ROWS, HC), jnp.float32),
            pltpu.VMEM((CHUNK, HC), jnp.float32),
            pltpu.VMEM((CHUNK, HC), jnp.float32),
            pltpu.VMEM((PROWS, BLK), jnp.int32),
            pltpu.VMEM((PROWS, BLK), jnp.int32),
            pltpu.VMEM((PROWS, BLK), jnp.int32),
            pltpu.VMEM((PROWS, BLK), jnp.int32),
            pltpu.SemaphoreType.DMA,
            pltpu.SemaphoreType.DMA,
            pltpu.SemaphoreType.DMA,
            pltpu.SemaphoreType.DMA,
        ],
        compiler_params=pltpu.CompilerParams(use_tc_tiling_on_sc=False),
    )
    def seg_kernel(xall_hbm, g_hbm2, s_hbm2, out_hbm,
                   acc, rowsA, rowsB, g0, s0, g1, s1,
                   gsem, ssemA, ssemB, isem):
        c = lax.axis_index("c")
        s = lax.axis_index("s")
        x_hbm = xall_hbm.at[c]

        for d in range(2):
            g_hbm = g_hbm2.at[d]
            s_hbm = s_hbm2.at[d]

            def idx_row0(b):
                return s * ROWS_PT + b * PROWS

            def fire_idx(b, gbuf, sbuf):
                pltpu.async_copy(g_hbm.at[pl.ds(idx_row0(b), PROWS)],
                                 gbuf, isem)
                pltpu.async_copy(s_hbm.at[pl.ds(idx_row0(b), PROWS)],
                                 sbuf, isem)

            def wait_idx(b, gbuf, sbuf):
                pltpu.make_async_copy(
                    g_hbm.at[pl.ds(idx_row0(b), PROWS)], gbuf, isem).wait()
                pltpu.make_async_copy(
                    s_hbm.at[pl.ds(idx_row0(b), PROWS)], sbuf, isem).wait()

            def drain_sb(sbuf):
                for u in range(STR):
                    pltpu.make_async_copy(
                        rowsB.at[pl.ds(u * BLK, BLK)],
                        acc.at[sbuf.at[STR + u]], ssemB).wait()

            def run_body(gbuf, sbuf):
                ga = [
                    pltpu.async_copy(x_hbm.at[gbuf.at[u]],
                                     rowsA.at[pl.ds(u * BLK, BLK)], gsem)
                    for u in range(STR)
                ]
                gb = [
                    pltpu.async_copy(x_hbm.at[gbuf.at[STR + u]],
                                     rowsB.at[pl.ds(u * BLK, BLK)], gsem)
                    for u in range(STR)
                ]
                for dd in ga:
                    dd.wait()
                sa = [
                    pltpu.async_copy(rowsA.at[pl.ds(u * BLK, BLK)],
                                     acc.at[sbuf.at[u]], ssemA, add=True)
                    for u in range(STR)
                ]
                for dd in gb:
                    dd.wait()
                for dd in sa:
                    dd.wait()
                for u in range(STR):
                    pltpu.async_copy(rowsB.at[pl.ds(u * BLK, BLK)],
                                     acc.at[sbuf.at[STR + u]],
                                     ssemB, add=True)

            def zrow(i, z):
                rowsA[i, pl.ds(0, 16)] = jnp.zeros((16,), jnp.float32)
                rowsA[i, pl.ds(16, 16)] = jnp.zeros((16,), jnp.float32)
                return z
            lax.fori_loop(0, CHUNK, zrow, 0)
            zbase = s * APT
            zdescs = []
            zoff = 0
            while zoff < APT:
                zn = min(CHUNK, APT - zoff)
                zdescs.append(pltpu.async_copy(
                    rowsA.at[pl.ds(0, zn)],
                    acc.at[pl.ds(zbase + zoff, zn)], gsem))
                zoff += zn
            for dd in zdescs:
                dd.wait()
            plsc.subcore_barrier()

            fire_idx(0, g0, s0)

            def body(tt, carry):
                b0 = 2 * tt
                wait_idx(b0, g0, s0)

                @pl.when(tt > 0)
                def _():
                    drain_sb(s1)

                fire_idx(b0 + 1, g1, s1)
                run_body(g0, s0)
                wait_idx(b0 + 1, g1, s1)
                drain_sb(s0)
                fire_idx(b0 + 2, g0, s0)
                run_body(g1, s1)
                return carry
            lax.fori_loop(0, NBODY // 2, body, 0)
            wait_idx(NBODY, g0, s0)
            drain_sb(s1)
            plsc.subcore_barrier()

            pltpu.sync_copy(
                acc.at[pl.ds(s * WPT, WPT)],
                out_hbm.at[d].at[pl.ds(c * NOUT + s * WPT, WPT)],
            )
            plsc.subcore_barrier()

    return seg_kernel(xflat, gidx4, sidx2)


def _mlp(out4, W1r, b1, W2, b2):
    B = 2000

    def body(a_ref, w1_ref, b1_ref, w2_ref, b2_ref, o_ref):
        h1 = (
            jnp.dot(a_ref[0, 0], w1_ref[0, 0], preferred_element_type=jnp.float32)
            + jnp.dot(a_ref[0, 1], w1_ref[0, 1], preferred_element_type=jnp.float32)
            + jnp.dot(a_ref[1, 0], w1_ref[1, 0], preferred_element_type=jnp.float32)
            + jnp.dot(a_ref[1, 1], w1_ref[1, 1], preferred_element_type=jnp.float32)
            + b1_ref[...]
        )
        h1 = jnp.maximum(h1, 0.0)
        o_ref[...] = (
            jnp.dot(h1, w2_ref[...], preferred_element_type=jnp.float32)
            + b2_ref[...]
        )

    return pl.pallas_call(
        body,
        grid=(N // B,),
        in_specs=[
            pl.BlockSpec((2, 2, B, HC), lambda i: (0, 0, i, 0)),
            pl.BlockSpec((2, 2, HC, H), lambda i: (0, 0, 0, 0)),
            pl.BlockSpec((1, H), lambda i: (0, 0)),
            pl.BlockSpec((H, C), lambda i: (0, 0)),
            pl.BlockSpec((1, C), lambda i: (0, 0)),
        ],
        out_specs=pl.BlockSpec((B, C), lambda i: (i, 0)),
        out_shape=jax.ShapeDtypeStruct((N, C), jnp.float32),
    )(out4, W1r, b1.reshape(1, H), W2, b2.reshape(1, C))


def kernel(x, edge_index, W1, b1, W2, b2):
    src = edge_index[0].astype(jnp.int32)
    dst = edge_index[1].astype(jnp.int32)
    npad = IDXAL * BLK - E
    gpad = jnp.zeros((npad,), jnp.int32)
    spad = jnp.full((npad,), N, jnp.int32)
    gidx2 = jnp.stack([jnp.concatenate([src, gpad]),
                       jnp.concatenate([dst, gpad])]).reshape(2, IDXAL, BLK)
    sidx2 = jnp.stack([jnp.concatenate([dst, spad]),
                       jnp.concatenate([src, spad])]).reshape(2, IDXAL, BLK)
    xall = jnp.stack([x[:, :HC], x[:, HC:]])
    out = _sc_segsum(xall, gidx2, sidx2)
    out4 = out.reshape(2, 2, NOUT, HC)
    return _mlp(out4, W1.reshape(2, 2, HC, H), b1, W2, b2)


# speedup vs baseline: 1.1795x; 1.0694x over previous
import functools

import jax
import jax.numpy as jnp
from jax import lax
from jax.experimental import pallas as pl
from jax.experimental.pallas import tpu as pltpu
from jax.experimental.pallas import tpu_sc as plsc

N = 50000
E = 800000
C = 64
HC = 32
H = 256
NC, NS = 2, 16
BLK = 128
STR = 3
CHUNK = BLK * STR
PAIR = 2 * CHUNK
PROWS = PAIR // BLK
NBODY = 66
EPT = NBODY * PAIR
EPAD = EPT * NS
IDXROWS = EPAD // BLK
IDXAL = IDXROWS + 8
ROWS_PT = IDXROWS // NS
ACC_ROWS = 50048
APT = ACC_ROWS // NS
NOUT = ACC_ROWS
WPT = NOUT // NS


def _sc_segsum(xflat, gidx4, sidx2):
    mesh = plsc.VectorSubcoreMesh(core_axis_name="c", subcore_axis_name="s")

    @functools.partial(
        pl.kernel,
        out_type=jax.ShapeDtypeStruct((2, 2 * NOUT, HC), jnp.float32),
        mesh=mesh,
        scratch_types=[
            pltpu.VMEM_SHARED((ACC_ROWS, HC), jnp.float32),
            pltpu.VMEM((CHUNK, HC), jnp.float32),
            pltpu.VMEM((CHUNK, HC), jnp.float32),
            pltpu.VMEM((PROWS, BLK), jnp.int32),
            pltpu.VMEM((PROWS, BLK), jnp.int32),
            pltpu.VMEM((PROWS, BLK), jnp.int32),
            pltpu.VMEM((PROWS, BLK), jnp.int32),
            pltpu.SemaphoreType.DMA,
            pltpu.SemaphoreType.DMA,
            pltpu.SemaphoreType.DMA,
            pltpu.SemaphoreType.DMA,
        ],
        compiler_params=pltpu.CompilerParams(use_tc_tiling_on_sc=False),
    )
    def seg_kernel(xall_hbm, g_hbm2, s_hbm2, out_hbm,
                   acc, rowsA, rowsB, g0, s0, g1, s1,
                   gsem, ssemA, ssemB, isem):
        c = lax.axis_index("c")
        s = lax.axis_index("s")
        x_hbm = xall_hbm.at[c]

        for d in range(2):
            g_hbm = g_hbm2.at[d]
            s_hbm = s_hbm2.at[d]

            def idx_row0(b):
                return s * ROWS_PT + b * PROWS

            def fire_idx(b, gbuf, sbuf):
                pltpu.async_copy(g_hbm.at[pl.ds(idx_row0(b), PROWS)],
                                 gbuf, isem)
                pltpu.async_copy(s_hbm.at[pl.ds(idx_row0(b), PROWS)],
                                 sbuf, isem)

            def wait_idx(b, gbuf, sbuf):
                pltpu.make_async_copy(
                    g_hbm.at[pl.ds(idx_row0(b), PROWS)], gbuf, isem).wait()
                pltpu.make_async_copy(
                    s_hbm.at[pl.ds(idx_row0(b), PROWS)], sbuf, isem).wait()

            def drain_sb(sbuf):
                for u in range(STR):
                    pltpu.make_async_copy(
                        rowsB.at[pl.ds(u * BLK, BLK)],
                        acc.at[sbuf.at[STR + u]], ssemB).wait()

            def fire_ga(gbuf):
                return [
                    pltpu.async_copy(x_hbm.at[gbuf.at[u]],
                                     rowsA.at[pl.ds(u * BLK, BLK)], gsem)
                    for u in range(STR)
                ]

            def finish_body(gbuf, sbuf, ga):
                gb = [
                    pltpu.async_copy(x_hbm.at[gbuf.at[STR + u]],
                                     rowsB.at[pl.ds(u * BLK, BLK)], gsem)
                    for u in range(STR)
                ]
                for dd in ga:
                    dd.wait()
                sa = [
                    pltpu.async_copy(rowsA.at[pl.ds(u * BLK, BLK)],
                                     acc.at[sbuf.at[u]], ssemA, add=True)
                    for u in range(STR)
                ]
                for dd in gb:
                    dd.wait()
                for dd in sa:
                    dd.wait()
                for u in range(STR):
                    pltpu.async_copy(rowsB.at[pl.ds(u * BLK, BLK)],
                                     acc.at[sbuf.at[STR + u]],
                                     ssemB, add=True)

            def zrow(i, z):
                rowsA[i, pl.ds(0, 16)] = jnp.zeros((16,), jnp.float32)
                rowsA[i, pl.ds(16, 16)] = jnp.zeros((16,), jnp.float32)
                return z
            lax.fori_loop(0, CHUNK, zrow, 0)
            zbase = s * APT
            zdescs = []
            zoff = 0
            while zoff < APT:
                zn = min(CHUNK, APT - zoff)
                zdescs.append(pltpu.async_copy(
                    rowsA.at[pl.ds(0, zn)],
                    acc.at[pl.ds(zbase + zoff, zn)], gsem))
                zoff += zn
            for dd in zdescs:
                dd.wait()
            plsc.subcore_barrier()

            fire_idx(0, g0, s0)

            def body(tt, carry):
                b0 = 2 * tt
                wait_idx(b0, g0, s0)
                ga0 = fire_ga(g0)

                @pl.when(tt > 0)
                def _():
                    drain_sb(s1)

                fire_idx(b0 + 1, g1, s1)
                finish_body(g0, s0, ga0)
                wait_idx(b0 + 1, g1, s1)
                ga1 = fire_ga(g1)
                drain_sb(s0)
                fire_idx(b0 + 2, g0, s0)
                finish_body(g1, s1, ga1)
                return carry
            lax.fori_loop(0, NBODY // 2, body, 0)
            wait_idx(NBODY, g0, s0)
            drain_sb(s1)
            plsc.subcore_barrier()

            pltpu.sync_copy(
                acc.at[pl.ds(s * WPT, WPT)],
                out_hbm.at[d].at[pl.ds(c * NOUT + s * WPT, WPT)],
            )
            plsc.subcore_barrier()

    return seg_kernel(xflat, gidx4, sidx2)


def _mlp(out4, W1r, b1, W2, b2):
    B = 2000

    def body(a_ref, w1_ref, b1_ref, w2_ref, b2_ref, o_ref):
        h1 = (
            jnp.dot(a_ref[0, 0], w1_ref[0, 0], preferred_element_type=jnp.float32)
            + jnp.dot(a_ref[0, 1], w1_ref[0, 1], preferred_element_type=jnp.float32)
            + jnp.dot(a_ref[1, 0], w1_ref[1, 0], preferred_element_type=jnp.float32)
            + jnp.dot(a_ref[1, 1], w1_ref[1, 1], preferred_element_type=jnp.float32)
            + b1_ref[...]
        )
        h1 = jnp.maximum(h1, 0.0)
        o_ref[...] = (
            jnp.dot(h1, w2_ref[...], preferred_element_type=jnp.float32)
            + b2_ref[...]
        )

    return pl.pallas_call(
        body,
        grid=(N // B,),
        in_specs=[
            pl.BlockSpec((2, 2, B, HC), lambda i: (0, 0, i, 0)),
            pl.BlockSpec((2, 2, HC, H), lambda i: (0, 0, 0, 0)),
            pl.BlockSpec((1, H), lambda i: (0, 0)),
            pl.BlockSpec((H, C), lambda i: (0, 0)),
            pl.BlockSpec((1, C), lambda i: (0, 0)),
        ],
        out_specs=pl.BlockSpec((B, C), lambda i: (i, 0)),
        out_shape=jax.ShapeDtypeStruct((N, C), jnp.float32),
    )(out4, W1r, b1.reshape(1, H), W2, b2.reshape(1, C))


def kernel(x, edge_index, W1, b1, W2, b2):
    src = edge_index[0].astype(jnp.int32)
    dst = edge_index[1].astype(jnp.int32)
    npad = IDXAL * BLK - E
    gpad = jnp.zeros((npad,), jnp.int32)
    spad = jnp.full((npad,), N, jnp.int32)
    gidx2 = jnp.stack([jnp.concatenate([src, gpad]),
                       jnp.concatenate([dst, gpad])]).reshape(2, IDXAL, BLK)
    sidx2 = jnp.stack([jnp.concatenate([dst, spad]),
                       jnp.concatenate([src, spad])]).reshape(2, IDXAL, BLK)
    xall = jnp.stack([x[:, :HC], x[:, HC:]])
    out = _sc_segsum(xall, gidx2, sidx2)
    out4 = out.reshape(2, 2, NOUT, HC)
    return _mlp(out4, W1.reshape(2, 2, HC, H), b1, W2, b2)


# speedup vs baseline: 1.1903x; 1.0092x over previous
import functools

import jax
import jax.numpy as jnp
from jax import lax
from jax.experimental import pallas as pl
from jax.experimental.pallas import tpu as pltpu
from jax.experimental.pallas import tpu_sc as plsc

N = 50000
E = 800000
C = 64
HC = 32
H = 256
NC, NS = 2, 16
BLK = 128
STR = 3
CHUNK = BLK * STR
PAIR = 2 * CHUNK
PROWS = PAIR // BLK
NBODY = 66
EPT = NBODY * PAIR
EPAD = EPT * NS
IDXROWS = EPAD // BLK
IDXAL = IDXROWS + 8
ROWS_PT = IDXROWS // NS
ACC_ROWS = 50048
APT = ACC_ROWS // NS
NOUT = ACC_ROWS
WPT = NOUT // NS


def _sc_segsum(xflat, gidx4, sidx2):
    mesh = plsc.VectorSubcoreMesh(core_axis_name="c", subcore_axis_name="s")

    @functools.partial(
        pl.kernel,
        out_type=jax.ShapeDtypeStruct((2, 2 * NOUT, HC), jnp.float32),
        mesh=mesh,
        scratch_types=[
            pltpu.VMEM_SHARED((ACC_ROWS, HC), jnp.float32),
            pltpu.VMEM((CHUNK, HC), jnp.float32),
            pltpu.VMEM((CHUNK, HC), jnp.float32),
            pltpu.VMEM((PROWS, BLK), jnp.int32),
            pltpu.VMEM((PROWS, BLK), jnp.int32),
            pltpu.VMEM((PROWS, BLK), jnp.int32),
            pltpu.VMEM((PROWS, BLK), jnp.int32),
            pltpu.SemaphoreType.DMA,
            pltpu.SemaphoreType.DMA,
            pltpu.SemaphoreType.DMA,
            pltpu.SemaphoreType.DMA,
        ],
        compiler_params=pltpu.CompilerParams(use_tc_tiling_on_sc=False),
    )
    def seg_kernel(xall_hbm, g_hbm2, s_hbm2, out_hbm,
                   acc, rowsA, rowsB, g0, s0, g1, s1,
                   gsem, ssemA, ssemB, isem):
        c = lax.axis_index("c")
        s = lax.axis_index("s")
        x_hbm = xall_hbm.at[c]

        for d in range(2):
            g_hbm = g_hbm2.at[d]
            s_hbm = s_hbm2.at[d]

            def idx_row0(b):
                return s * ROWS_PT + b * PROWS

            def fire_idx(b, gbuf, sbuf):
                pltpu.async_copy(g_hbm.at[pl.ds(idx_row0(b), PROWS)],
                                 gbuf, isem)
                pltpu.async_copy(s_hbm.at[pl.ds(idx_row0(b), PROWS)],
                                 sbuf, isem)

            def wait_idx(b, gbuf, sbuf):
                pltpu.make_async_copy(
                    g_hbm.at[pl.ds(idx_row0(b), PROWS)], gbuf, isem).wait()
                pltpu.make_async_copy(
                    s_hbm.at[pl.ds(idx_row0(b), PROWS)], sbuf, isem).wait()

            def drain_sb(sbuf):
                for u in range(STR):
                    pltpu.make_async_copy(
                        rowsB.at[pl.ds(u * BLK, BLK)],
                        acc.at[sbuf.at[STR + u]], ssemB).wait()

            def fire_ga(gbuf):
                return [
                    pltpu.async_copy(x_hbm.at[gbuf.at[u]],
                                     rowsA.at[pl.ds(u * BLK, BLK)], gsem)
                    for u in range(STR)
                ]

            def finish_body(gbuf, sbuf, ga):
                gb = [
                    pltpu.async_copy(x_hbm.at[gbuf.at[STR + u]],
                                     rowsB.at[pl.ds(u * BLK, BLK)], gsem)
                    for u in range(STR)
                ]
                for dd in ga:
                    dd.wait()
                sa = [
                    pltpu.async_copy(rowsA.at[pl.ds(u * BLK, BLK)],
                                     acc.at[sbuf.at[u]], ssemA, add=True)
                    for u in range(STR)
                ]
                for dd in gb:
                    dd.wait()
                for dd in sa:
                    dd.wait()
                for u in range(STR):
                    pltpu.async_copy(rowsB.at[pl.ds(u * BLK, BLK)],
                                     acc.at[sbuf.at[STR + u]],
                                     ssemB, add=True)

            def zrow(i, z):
                rowsA[i, pl.ds(0, 16)] = jnp.zeros((16,), jnp.float32)
                rowsA[i, pl.ds(16, 16)] = jnp.zeros((16,), jnp.float32)
                return z
            lax.fori_loop(0, CHUNK, zrow, 0)
            zbase = s * APT
            zdescs = []
            zoff = 0
            while zoff < APT:
                zn = min(CHUNK, APT - zoff)
                zdescs.append(pltpu.async_copy(
                    rowsA.at[pl.ds(0, zn)],
                    acc.at[pl.ds(zbase + zoff, zn)], gsem))
                zoff += zn
            for dd in zdescs:
                dd.wait()
            plsc.subcore_barrier()

            fire_idx(0, g0, s0)

            def body(tt, carry):
                b0 = 2 * tt
                wait_idx(b0, g0, s0)
                ga0 = fire_ga(g0)

                @pl.when(tt > 0)
                def _():
                    drain_sb(s1)

                fire_idx(b0 + 1, g1, s1)
                finish_body(g0, s0, ga0)
                wait_idx(b0 + 1, g1, s1)
                ga1 = fire_ga(g1)
                drain_sb(s0)
                fire_idx(b0 + 2, g0, s0)
                finish_body(g1, s1, ga1)
                return carry
            lax.fori_loop(0, NBODY // 2, body, 0)
            wait_idx(NBODY, g0, s0)
            drain_sb(s1)
            plsc.subcore_barrier()

            pltpu.sync_copy(
                acc.at[pl.ds(s * WPT, WPT)],
                out_hbm.at[d].at[pl.ds(c * NOUT + s * WPT, WPT)],
            )
            plsc.subcore_barrier()

    return seg_kernel(xflat, gidx4, sidx2)


def _mlp(out4, W1r, b1, W2, b2):
    B = 5000

    def body(a_ref, w1_ref, b1_ref, w2_ref, b2_ref, o_ref):
        h1 = (
            jnp.dot(a_ref[0, 0], w1_ref[0, 0], preferred_element_type=jnp.float32)
            + jnp.dot(a_ref[0, 1], w1_ref[0, 1], preferred_element_type=jnp.float32)
            + jnp.dot(a_ref[1, 0], w1_ref[1, 0], preferred_element_type=jnp.float32)
            + jnp.dot(a_ref[1, 1], w1_ref[1, 1], preferred_element_type=jnp.float32)
            + b1_ref[...]
        )
        h1 = jnp.maximum(h1, 0.0)
        o_ref[...] = (
            jnp.dot(h1, w2_ref[...], preferred_element_type=jnp.float32)
            + b2_ref[...]
        )

    return pl.pallas_call(
        body,
        grid=(N // B,),
        in_specs=[
            pl.BlockSpec((2, 2, B, HC), lambda i: (0, 0, i, 0)),
            pl.BlockSpec((2, 2, HC, H), lambda i: (0, 0, 0, 0)),
            pl.BlockSpec((1, H), lambda i: (0, 0)),
            pl.BlockSpec((H, C), lambda i: (0, 0)),
            pl.BlockSpec((1, C), lambda i: (0, 0)),
        ],
        out_specs=pl.BlockSpec((B, C), lambda i: (i, 0)),
        out_shape=jax.ShapeDtypeStruct((N, C), jnp.float32),
    )(out4, W1r, b1.reshape(1, H), W2, b2.reshape(1, C))


def kernel(x, edge_index, W1, b1, W2, b2):
    src = edge_index[0].astype(jnp.int32)
    dst = edge_index[1].astype(jnp.int32)
    npad = IDXAL * BLK - E
    gpad = jnp.zeros((npad,), jnp.int32)
    spad = jnp.full((npad,), N, jnp.int32)
    gidx2 = jnp.stack([jnp.concatenate([src, gpad]),
                       jnp.concatenate([dst, gpad])]).reshape(2, IDXAL, BLK)
    sidx2 = jnp.stack([jnp.concatenate([dst, spad]),
                       jnp.concatenate([src, spad])]).reshape(2, IDXAL, BLK)
    xall = jnp.stack([x[:, :HC], x[:, HC:]])
    out = _sc_segsum(xall, gidx2, sidx2)
    out4 = out.reshape(2, 2, NOUT, HC)
    return _mlp(out4, W1.reshape(2, 2, HC, H), b1, W2, b2)
